# Initial kernel scaffold; baseline (speedup 1.0000x reference)
import functools

import jax
import jax.numpy as jnp
from jax import lax
from jax.experimental import pallas as pl
from jax.experimental.pallas import tpu as pltpu
from jax.experimental.pallas import tpu_sc as plsc

N_NODES = 10000
D = 128
N_EDGES = 320000
NC = 2
NS = 16
NW = NC * NS
EDGES_PER_W = N_EDGES // NW
CHUNK = 80
NCHUNK = EDGES_PER_W // CHUNK
ROWS_PER_TILE = N_NODES // NS
ZROWS = 125

_MESH = plsc.VectorSubcoreMesh(core_axis_name="c", subcore_axis_name="s")


@functools.partial(
    pl.kernel,
    out_type=jax.ShapeDtypeStruct((NW, N_NODES), jnp.float32),
    mesh=_MESH,
    scratch_types=[
        pltpu.VMEM((NCHUNK, CHUNK), jnp.int32),
        pltpu.VMEM((N_NODES,), jnp.float32),
    ],
)
def _deg_kernel(dst_hbm, out_hbm, idx_d, degp):
    c = lax.axis_index("c")
    s = lax.axis_index("s")
    w = c * NS + s

    def zero(i, carry):
        degp[pl.ds(i * 16, 16)] = jnp.zeros((16,), jnp.float32)
        return carry

    lax.fori_loop(0, N_NODES // 16, zero, 0)

    pltpu.sync_copy(dst_hbm.at[w], idx_d)
    ones = jnp.ones((16,), jnp.float32)

    def body(j, carry):
        for k in range(CHUNK // 16):
            idx16 = idx_d[j, pl.ds(k * 16, 16)]
            plsc.addupdate_scatter(degp, [idx16], ones)
        return carry

    lax.fori_loop(0, NCHUNK, body, 0)
    pltpu.sync_copy(degp, out_hbm.at[w])


@functools.partial(
    pl.kernel,
    out_type=jax.ShapeDtypeStruct((NC, N_NODES, D), jnp.float32),
    mesh=_MESH,
    scratch_types=[
        pltpu.VMEM((NCHUNK, CHUNK), jnp.int32),
        pltpu.VMEM((NCHUNK, CHUNK), jnp.int32),
        pltpu.VMEM((CHUNK, D), jnp.float32),
        pltpu.VMEM((ZROWS, D), jnp.float32),
        pltpu.VMEM_SHARED((N_NODES, D), jnp.float32),
        pltpu.SemaphoreType.DMA,
    ],
)
def _scatter_kernel(src_hbm, dst_hbm, g_hbm, out_hbm,
                    idx_s, idx_d, rows, zbuf, acc, sem):
    c = lax.axis_index("c")
    s = lax.axis_index("s")
    w = c * NS + s

    def zrow(i, carry):
        def zcol(k, inner):
            zbuf[i, pl.ds(k * 16, 16)] = jnp.zeros((16,), jnp.float32)
            return inner
        return lax.fori_loop(0, D // 16, zcol, carry)

    lax.fori_loop(0, ZROWS, zrow, 0)
    for z in range(ROWS_PER_TILE // ZROWS):
        pltpu.sync_copy(
            zbuf, acc.at[pl.ds(s * ROWS_PER_TILE + z * ZROWS, ZROWS)])
    plsc.subcore_barrier()

    pltpu.sync_copy(src_hbm.at[w], idx_s)
    pltpu.sync_copy(dst_hbm.at[w], idx_d)

    def body(j, carry):
        pltpu.async_copy(g_hbm.at[idx_s.at[j]], rows, sem).wait()
        pltpu.sync_copy(rows, acc.at[idx_d.at[j]], add=True)
        return carry

    lax.fori_loop(0, NCHUNK, body, 0)
    plsc.subcore_barrier()
    pltpu.sync_copy(acc.at[pl.ds(s * ROWS_PER_TILE, ROWS_PER_TILE)],
                    out_hbm.at[c, pl.ds(s * ROWS_PER_TILE, ROWS_PER_TILE)])


def _tc1_body(degpt_ref, x_ref, w1_ref, dis_ref, g1_ref):
    deg = jnp.sum(degpt_ref[...], axis=1, keepdims=True) + 1.0
    dis = lax.rsqrt(deg)
    dis_ref[...] = dis
    h = jnp.dot(x_ref[...], w1_ref[...],
                preferred_element_type=jnp.float32,
                precision=lax.Precision.HIGHEST)
    g1_ref[...] = h * dis


def _tc2_body(s1_ref, g1_ref, dis_ref, b1_ref, w2_ref, g2_ref):
    dis = dis_ref[...]
    t = (s1_ref[0] + s1_ref[1] + g1_ref[...]) * dis + b1_ref[...]
    t = jnp.maximum(t, 0.0)
    h2 = jnp.dot(t, w2_ref[...],
                 preferred_element_type=jnp.float32,
                 precision=lax.Precision.HIGHEST)
    g2_ref[...] = h2 * dis


def _tc3_body(s2_ref, g2_ref, dis_ref, b2_ref, out_ref):
    out_ref[...] = ((s2_ref[0] + s2_ref[1] + g2_ref[...]) * dis_ref[...]
                    + b2_ref[...])


_tc1 = pl.pallas_call(
    _tc1_body,
    out_shape=[jax.ShapeDtypeStruct((N_NODES, 1), jnp.float32),
               jax.ShapeDtypeStruct((N_NODES, D), jnp.float32)],
)

_tc2 = pl.pallas_call(
    _tc2_body,
    out_shape=jax.ShapeDtypeStruct((N_NODES, D), jnp.float32),
)

_tc3 = pl.pallas_call(
    _tc3_body,
    out_shape=jax.ShapeDtypeStruct((N_NODES, D), jnp.float32),
)


def kernel(x, edge_index, W1, b1, W2, b2):
    src = edge_index[0].astype(jnp.int32).reshape(NW, NCHUNK, CHUNK)
    dst = edge_index[1].astype(jnp.int32).reshape(NW, NCHUNK, CHUNK)
    b1r = b1.reshape(1, D)
    b2r = b2.reshape(1, D)

    degp = _deg_kernel(dst)
    dis, g1 = _tc1(degp.T, x, W1)
    s1 = _scatter_kernel(src, dst, g1)
    g2 = _tc2(s1, g1, dis, b1r, W2)
    s2 = _scatter_kernel(src, dst, g2)
    return _tc3(s2, g2, dis, b2r)


# speedup vs baseline: 20.3297x; 20.3297x over previous
import functools

import jax
import jax.numpy as jnp
from jax import lax
from jax.experimental import pallas as pl
from jax.experimental.pallas import tpu as pltpu
from jax.experimental.pallas import tpu_sc as plsc

N_NODES = 10000
D = 128
N_EDGES = 320000
NC = 2
NS = 16
NW = NC * NS
EDGES_PER_W = N_EDGES // NW
CHUNK = 80
NCHUNK = EDGES_PER_W // CHUNK
N_PAD = 10240
ROWS_PER_TILE = N_PAD // NS

_MESH = plsc.VectorSubcoreMesh(core_axis_name="c", subcore_axis_name="s")


@functools.partial(
    pl.kernel,
    out_type=jax.ShapeDtypeStruct((NW, N_NODES), jnp.float32),
    mesh=_MESH,
    scratch_types=[
        pltpu.VMEM((NCHUNK, CHUNK), jnp.int32),
        pltpu.VMEM((N_NODES,), jnp.float32),
    ],
    compiler_params=pltpu.CompilerParams(needs_layout_passes=False),
)
def _deg_kernel(dst_hbm, out_hbm, idx_d, degp):
    c = lax.axis_index("c")
    s = lax.axis_index("s")
    w = c * NS + s

    def zero(i, carry):
        degp[pl.ds(i * 16, 16)] = jnp.zeros((16,), jnp.float32)
        return carry

    lax.fori_loop(0, N_NODES // 16, zero, 0)

    pltpu.sync_copy(dst_hbm.at[w], idx_d)
    ones = jnp.ones((16,), jnp.float32)

    def body(j, carry):
        for k in range(CHUNK // 16):
            idx16 = idx_d[j, pl.ds(k * 16, 16)]
            plsc.addupdate_scatter(degp, [idx16], ones)
        return carry

    lax.fori_loop(0, NCHUNK, body, 0)
    pltpu.sync_copy(degp, out_hbm.at[w])


@functools.partial(
    pl.kernel,
    out_type=jax.ShapeDtypeStruct((NC, N_PAD, D), jnp.float32),
    mesh=_MESH,
    scratch_types=[
        pltpu.VMEM((NCHUNK, CHUNK), jnp.int32),
        pltpu.VMEM((NCHUNK, CHUNK), jnp.int32),
        pltpu.VMEM((CHUNK, D), jnp.float32),
        pltpu.VMEM_SHARED((N_PAD, D), jnp.float32),
        pltpu.SemaphoreType.DMA,
    ],
)
def _scatter_kernel(src_hbm, dst_hbm, g_hbm, out_hbm,
                    idx_s, idx_d, rows, acc, sem):
    c = lax.axis_index("c")
    s = lax.axis_index("s")
    w = c * NS + s

    def zrow(i, carry):
        def zcol(k, inner):
            rows[i, pl.ds(k * 16, 16)] = jnp.zeros((16,), jnp.float32)
            return inner
        return lax.fori_loop(0, D // 16, zcol, carry)

    lax.fori_loop(0, CHUNK, zrow, 0)
    for z in range(ROWS_PER_TILE // CHUNK):
        pltpu.sync_copy(
            rows, acc.at[pl.ds(s * ROWS_PER_TILE + z * CHUNK, CHUNK)])
    plsc.subcore_barrier()

    pltpu.sync_copy(src_hbm.at[w], idx_s)
    pltpu.sync_copy(dst_hbm.at[w], idx_d)

    def body(j, carry):
        pltpu.async_copy(g_hbm.at[idx_s.at[j]], rows, sem).wait()
        pltpu.sync_copy(rows, acc.at[idx_d.at[j]], add=True)
        return carry

    lax.fori_loop(0, NCHUNK, body, 0)
    plsc.subcore_barrier()
    pltpu.sync_copy(acc.at[pl.ds(s * ROWS_PER_TILE, ROWS_PER_TILE)],
                    out_hbm.at[c, pl.ds(s * ROWS_PER_TILE, ROWS_PER_TILE)])


def _tc1_body(degpt_ref, x_ref, w1_ref, dis_ref, g1_ref):
    deg = jnp.sum(degpt_ref[...], axis=1, keepdims=True) + 1.0
    dis = lax.rsqrt(deg)
    dis_ref[...] = dis
    h = jnp.dot(x_ref[...], w1_ref[...],
                preferred_element_type=jnp.float32,
                precision=lax.Precision.HIGHEST)
    g1_ref[...] = h * dis


def _tc2_body(s1_ref, g1_ref, dis_ref, b1_ref, w2_ref, g2_ref):
    dis = dis_ref[...]
    ssum = (s1_ref[0] + s1_ref[1])[:N_NODES]
    t = (ssum + g1_ref[...]) * dis + b1_ref[...]
    t = jnp.maximum(t, 0.0)
    h2 = jnp.dot(t, w2_ref[...],
                 preferred_element_type=jnp.float32,
                 precision=lax.Precision.HIGHEST)
    g2_ref[...] = h2 * dis


def _tc3_body(s2_ref, g2_ref, dis_ref, b2_ref, out_ref):
    ssum = (s2_ref[0] + s2_ref[1])[:N_NODES]
    out_ref[...] = (ssum + g2_ref[...]) * dis_ref[...] + b2_ref[...]


_tc1 = pl.pallas_call(
    _tc1_body,
    out_shape=[jax.ShapeDtypeStruct((N_NODES, 1), jnp.float32),
               jax.ShapeDtypeStruct((N_NODES, D), jnp.float32)],
)

_tc2 = pl.pallas_call(
    _tc2_body,
    out_shape=jax.ShapeDtypeStruct((N_NODES, D), jnp.float32),
)

_tc3 = pl.pallas_call(
    _tc3_body,
    out_shape=jax.ShapeDtypeStruct((N_NODES, D), jnp.float32),
)


def kernel(x, edge_index, W1, b1, W2, b2):
    src = edge_index[0].astype(jnp.int32).reshape(NW, NCHUNK, CHUNK)
    dst = edge_index[1].astype(jnp.int32).reshape(NW, NCHUNK, CHUNK)
    b1r = b1.reshape(1, D)
    b2r = b2.reshape(1, D)

    degp = _deg_kernel(dst)
    dis, g1 = _tc1(degp.T, x, W1)
    s1 = _scatter_kernel(src, dst, g1)
    g2 = _tc2(s1, g1, dis, b1r, W2)
    s2 = _scatter_kernel(src, dst, g2)
    return _tc3(s2, g2, dis, b2r)
